# Initial kernel scaffold; baseline (speedup 1.0000x reference)
import jax
import jax.numpy as jnp
from jax.experimental import pallas as pl


def kernel(x, edge_index, Wl1, bl1, Wr1, br1, att1, bias1, Wl2, bl2, Wr2, br2, att2, bias2, Wp, bp):
    raise NotImplementedError("write your pallas kernel here")


# speedup vs baseline: 13.5524x; 13.5524x over previous
import functools

import jax
import jax.numpy as jnp
from jax import lax
from jax.experimental import pallas as pl
from jax.experimental.pallas import tpu as pltpu
from jax.experimental.pallas import tpu_sc as plsc

N = 10000
E = 160000
D = 256
NC = 2
NS = 16
CW = 128
CH = 128
NCHUNK = E // CH


def _dual_matmul(x, Wl, bl, Wr, br, bm=1000):
    m, k = x.shape
    n = Wl.shape[1]

    def body(x_ref, wl_ref, blr, wr_ref, brr, ol_ref, or_ref):
        xb = x_ref[...]
        ol_ref[...] = jnp.dot(xb, wl_ref[...], preferred_element_type=jnp.float32) + blr[...]
        or_ref[...] = jnp.dot(xb, wr_ref[...], preferred_element_type=jnp.float32) + brr[...]

    return pl.pallas_call(
        body,
        grid=(m // bm,),
        in_specs=[
            pl.BlockSpec((bm, k), lambda i: (i, 0)),
            pl.BlockSpec((k, n), lambda i: (0, 0)),
            pl.BlockSpec((1, n), lambda i: (0, 0)),
            pl.BlockSpec((k, n), lambda i: (0, 0)),
            pl.BlockSpec((1, n), lambda i: (0, 0)),
        ],
        out_specs=[
            pl.BlockSpec((bm, n), lambda i: (i, 0)),
            pl.BlockSpec((bm, n), lambda i: (i, 0)),
        ],
        out_shape=[
            jax.ShapeDtypeStruct((m, n), jnp.float32),
            jax.ShapeDtypeStruct((m, n), jnp.float32),
        ],
    )(x, Wl, bl.reshape(1, n), Wr, br.reshape(1, n))


def _edge_scores(gl, gr, att_row, heads, be=2000):
    c = D // heads

    def body(gl_ref, gr_ref, att_ref, pw_ref, aux_ref):
        glb = gl_ref[...]
        z = glb + gr_ref[...]
        z = jnp.where(z >= 0, z, 0.2 * z)
        prod = z * att_ref[...]
        exps = []
        parts = []
        for h in range(heads):
            eh = jnp.exp(jnp.sum(prod[:, h * c:(h + 1) * c], axis=1, keepdims=True))
            exps.append(eh)
            parts.append(jnp.broadcast_to(eh, (be, c)))
        wex = jnp.concatenate(parts, axis=1) if heads > 1 else parts[0]
        m = wex * glb
        pw_ref[0] = m[:, :CW]
        pw_ref[1] = m[:, CW:]
        aux_ref[...] = jnp.concatenate(
            exps + [jnp.zeros((be, CW - heads), jnp.float32)], axis=1)

    return pl.pallas_call(
        body,
        grid=(E // be,),
        in_specs=[
            pl.BlockSpec((be, D), lambda i: (i, 0)),
            pl.BlockSpec((be, D), lambda i: (i, 0)),
            pl.BlockSpec((1, D), lambda i: (0, 0)),
        ],
        out_specs=[
            pl.BlockSpec((2, be, CW), lambda i: (0, i, 0)),
            pl.BlockSpec((be, CW), lambda i: (i, 0)),
        ],
        out_shape=[
            jax.ShapeDtypeStruct((2, E, CW), jnp.float32),
            jax.ShapeDtypeStruct((E, CW), jnp.float32),
        ],
    )(gl, gr, att_row.reshape(1, D))


def _softmax_elu(sm_ref, dn_ref, bias_ref, heads, bm):
    S = jnp.concatenate([sm_ref[0], sm_ref[1]], axis=1)
    dn = dn_ref[0] + dn_ref[1]
    c = D // heads
    dnb = jnp.concatenate(
        [jnp.broadcast_to(dn[:, h:h + 1], (bm, c)) for h in range(heads)],
        axis=1) if heads > 1 else jnp.broadcast_to(dn[:, 0:1], (bm, D))
    a = S / (dnb + 1e-16) + bias_ref[...]
    return jnp.where(a > 0, a, jnp.exp(a) - 1.0)


def _combine_matmul(SM, DN, bias, Wl, bl, Wr, br, heads, bm=1000):
    n = Wl.shape[1]

    def body(sm_ref, dn_ref, bias_ref, wl_ref, blr, wr_ref, brr, ol_ref, or_ref):
        h = _softmax_elu(sm_ref, dn_ref, bias_ref, heads, bm)
        ol_ref[...] = jnp.dot(h, wl_ref[...], preferred_element_type=jnp.float32) + blr[...]
        or_ref[...] = jnp.dot(h, wr_ref[...], preferred_element_type=jnp.float32) + brr[...]

    return pl.pallas_call(
        body,
        grid=(N // bm,),
        in_specs=[
            pl.BlockSpec((2, bm, CW), lambda i: (0, i, 0)),
            pl.BlockSpec((2, bm, CW), lambda i: (0, i, 0)),
            pl.BlockSpec((1, D), lambda i: (0, 0)),
            pl.BlockSpec((D, n), lambda i: (0, 0)),
            pl.BlockSpec((1, n), lambda i: (0, 0)),
            pl.BlockSpec((D, n), lambda i: (0, 0)),
            pl.BlockSpec((1, n), lambda i: (0, 0)),
        ],
        out_specs=[
            pl.BlockSpec((bm, n), lambda i: (i, 0)),
            pl.BlockSpec((bm, n), lambda i: (i, 0)),
        ],
        out_shape=[
            jax.ShapeDtypeStruct((N, n), jnp.float32),
            jax.ShapeDtypeStruct((N, n), jnp.float32),
        ],
    )(SM, DN, bias.reshape(1, D), Wl, bl.reshape(1, n), Wr, br.reshape(1, n))


def _combine_final(SM, DN, bias, Wp, bp, heads=1, bm=1000):
    n = Wp.shape[1]

    def body(sm_ref, dn_ref, bias_ref, w_ref, br_, o_ref):
        h = _softmax_elu(sm_ref, dn_ref, bias_ref, heads, bm)
        o_ref[...] = jnp.dot(h, w_ref[...], preferred_element_type=jnp.float32) + br_[...]

    return pl.pallas_call(
        body,
        grid=(N // bm,),
        in_specs=[
            pl.BlockSpec((2, bm, CW), lambda i: (0, i, 0)),
            pl.BlockSpec((2, bm, CW), lambda i: (0, i, 0)),
            pl.BlockSpec((1, D), lambda i: (0, 0)),
            pl.BlockSpec((D, n), lambda i: (0, 0)),
            pl.BlockSpec((1, n), lambda i: (0, 0)),
        ],
        out_specs=pl.BlockSpec((bm, n), lambda i: (i, 0)),
        out_shape=jax.ShapeDtypeStruct((N, n), jnp.float32),
    )(SM, DN, bias.reshape(1, D), Wp, bp.reshape(1, n))


def _sc_gather2(xl, xr, src, dst):
    mesh = plsc.VectorSubcoreMesh(core_axis_name="c", subcore_axis_name="s")
    NW = NC * NS
    iters = (NCHUNK + NW - 1) // NW

    @functools.partial(
        pl.kernel,
        mesh=mesh,
        out_type=[
            jax.ShapeDtypeStruct((E, D), jnp.float32),
            jax.ShapeDtypeStruct((E, D), jnp.float32),
        ],
        scratch_types=[
            pltpu.VMEM((CH,), jnp.int32),
            pltpu.VMEM((CH,), jnp.int32),
            pltpu.VMEM((CH, D), jnp.float32),
            pltpu.VMEM((CH, D), jnp.float32),
            pltpu.SemaphoreType.DMA,
            pltpu.SemaphoreType.DMA,
        ],
    )
    def k(xl_hbm, xr_hbm, src_hbm, dst_hbm, gl_hbm, gr_hbm,
          ibs, ibd, rs, rd, sem_a, sem_b):
        wid = lax.axis_index("s") * NC + lax.axis_index("c")

        def body(i, carry):
            chunk = i * NW + wid

            @pl.when(chunk < NCHUNK)
            def _():
                base = pl.multiple_of(chunk * CH, 8)
                pltpu.sync_copy(src_hbm.at[pl.ds(base, CH)], ibs)
                pltpu.sync_copy(dst_hbm.at[pl.ds(base, CH)], ibd)
                ca = pltpu.async_copy(xl_hbm.at[ibs], rs, sem_a)
                cb = pltpu.async_copy(xr_hbm.at[ibd], rd, sem_b)
                ca.wait()
                cb.wait()
                pltpu.sync_copy(rs, gl_hbm.at[pl.ds(base, CH)])
                pltpu.sync_copy(rd, gr_hbm.at[pl.ds(base, CH)])

            return carry

        lax.fori_loop(0, iters, body, 0)

    return k(xl, xr, src, dst)


def _sc_seg_sum(vals2, aux, idx):
    mesh = plsc.VectorSubcoreMesh(core_axis_name="c", subcore_axis_name="s")
    iters = (NCHUNK + NS - 1) // NS
    HALF = NCHUNK // NC
    hiters = (HALF + NS - 1) // NS
    BR = 80
    NB = N // BR
    biters = (NB + NS - 1) // NS

    @functools.partial(
        pl.kernel,
        mesh=mesh,
        out_type=[
            jax.ShapeDtypeStruct((NC, N, CW), jnp.float32),
            jax.ShapeDtypeStruct((NC, N, CW), jnp.float32),
        ],
        scratch_types=[
            pltpu.VMEM((CH,), jnp.int32),
            pltpu.VMEM((CH, CW), jnp.float32),
            pltpu.VMEM_SHARED((N, CW), jnp.float32),
            pltpu.SemaphoreType.DMA,
        ],
    )
    def k(vals_hbm, aux_hbm, idx_hbm, zeros_hbm, om_hbm, od_hbm,
          ibuf, vbuf, acc, sem):
        cid = lax.axis_index("c")
        sid = lax.axis_index("s")

        def zero_acc():
            def zero(j, carry):
                blk = j * NS + sid

                @pl.when(blk < NB)
                def _():
                    r0 = pl.multiple_of(blk * BR, 8)
                    pltpu.sync_copy(zeros_hbm, acc.at[pl.ds(r0, BR)])

                return carry

            lax.fori_loop(0, biters, zero, 0)

        def drain(dst_hbm):
            def d(j, carry):
                blk = j * NS + sid

                @pl.when(blk < NB)
                def _():
                    r0 = pl.multiple_of(blk * BR, 8)
                    pltpu.sync_copy(acc.at[pl.ds(r0, BR)],
                                    dst_hbm.at[cid].at[pl.ds(r0, BR)])

                return carry

            lax.fori_loop(0, biters, d, 0)

        def add_chunk(src_hbm, chunk):
            base = pl.multiple_of(chunk * CH, 8)
            pltpu.sync_copy(idx_hbm.at[pl.ds(base, CH)], ibuf)
            pltpu.sync_copy(src_hbm.at[pl.ds(base, CH)], vbuf)
            pltpu.sync_copy(vbuf, acc.at[ibuf], add=True)

        zero_acc()
        plsc.subcore_barrier()

        def body_m(i, carry):
            chunk = i * NS + sid

            @pl.when(chunk < NCHUNK)
            def _():
                add_chunk(vals_hbm.at[cid], chunk)

            return carry

        lax.fori_loop(0, iters, body_m, 0)
        plsc.subcore_barrier()
        drain(om_hbm)
        plsc.subcore_barrier()

        zero_acc()
        plsc.subcore_barrier()

        def body_d(i, carry):
            local = i * NS + sid

            @pl.when(local < HALF)
            def _():
                add_chunk(aux_hbm, cid * HALF + local)

            return carry

        lax.fori_loop(0, hiters, body_d, 0)
        plsc.subcore_barrier()
        drain(od_hbm)

    return k(vals2, aux, idx, jnp.zeros((BR, CW), jnp.float32))


def kernel(x, edge_index, Wl1, bl1, Wr1, br1, att1, bias1,
           Wl2, bl2, Wr2, br2, att2, bias2, Wp, bp):
    src = edge_index[0]
    dst = edge_index[1]

    xl1, xr1 = _dual_matmul(x, Wl1, bl1, Wr1, br1)
    gl1, gr1 = _sc_gather2(xl1, xr1, src, dst)
    PW1, AUX1 = _edge_scores(gl1, gr1, att1.reshape(-1), heads=4)
    SM1, DN1 = _sc_seg_sum(PW1, AUX1, dst)

    xl2, xr2 = _combine_matmul(SM1, DN1, bias1, Wl2, bl2, Wr2, br2, heads=4)
    gl2, gr2 = _sc_gather2(xl2, xr2, src, dst)
    PW2, AUX2 = _edge_scores(gl2, gr2, att2.reshape(-1), heads=1)
    SM2, DN2 = _sc_seg_sum(PW2, AUX2, dst)

    return _combine_final(SM2, DN2, bias2, Wp, bp)


# speedup vs baseline: 16.8316x; 1.2420x over previous
import functools

import jax
import jax.numpy as jnp
from jax import lax
from jax.experimental import pallas as pl
from jax.experimental.pallas import tpu as pltpu
from jax.experimental.pallas import tpu_sc as plsc

N = 10000
E = 160000
D = 256
NC = 2
NS = 16
CW = 128
CH = 128
NCHUNK = E // CH


def _dual_matmul(x, Wl, bl, Wr, br, bm=1000):
    m, k = x.shape
    n = Wl.shape[1]

    def body(x_ref, wl_ref, blr, wr_ref, brr, ol_ref, or_ref):
        xb = x_ref[...]
        ol_ref[...] = jnp.dot(xb, wl_ref[...], preferred_element_type=jnp.float32) + blr[...]
        or_ref[...] = jnp.dot(xb, wr_ref[...], preferred_element_type=jnp.float32) + brr[...]

    return pl.pallas_call(
        body,
        grid=(m // bm,),
        in_specs=[
            pl.BlockSpec((bm, k), lambda i: (i, 0)),
            pl.BlockSpec((k, n), lambda i: (0, 0)),
            pl.BlockSpec((1, n), lambda i: (0, 0)),
            pl.BlockSpec((k, n), lambda i: (0, 0)),
            pl.BlockSpec((1, n), lambda i: (0, 0)),
        ],
        out_specs=[
            pl.BlockSpec((bm, n), lambda i: (i, 0)),
            pl.BlockSpec((bm, n), lambda i: (i, 0)),
        ],
        out_shape=[
            jax.ShapeDtypeStruct((m, n), jnp.float32),
            jax.ShapeDtypeStruct((m, n), jnp.float32),
        ],
    )(x, Wl, bl.reshape(1, n), Wr, br.reshape(1, n))


def _edge_scores(gl, gr, att_row, heads, be=2000):
    c = D // heads

    def body(gl_ref, gr_ref, att_ref, pw_ref, aux_ref):
        glb = gl_ref[...]
        z = glb + gr_ref[...]
        z = jnp.where(z >= 0, z, 0.2 * z)
        prod = z * att_ref[...]
        exps = []
        parts = []
        for h in range(heads):
            eh = jnp.exp(jnp.sum(prod[:, h * c:(h + 1) * c], axis=1, keepdims=True))
            exps.append(eh)
            parts.append(jnp.broadcast_to(eh, (be, c)))
        wex = jnp.concatenate(parts, axis=1) if heads > 1 else parts[0]
        m = wex * glb
        pw_ref[0] = m[:, :CW]
        pw_ref[1] = m[:, CW:]
        aux_ref[...] = jnp.concatenate(
            exps + [jnp.zeros((be, CW - heads), jnp.float32)], axis=1)

    return pl.pallas_call(
        body,
        grid=(E // be,),
        in_specs=[
            pl.BlockSpec((be, D), lambda i: (i, 0)),
            pl.BlockSpec((be, D), lambda i: (i, 0)),
            pl.BlockSpec((1, D), lambda i: (0, 0)),
        ],
        out_specs=[
            pl.BlockSpec((2, be, CW), lambda i: (0, i, 0)),
            pl.BlockSpec((be, CW), lambda i: (i, 0)),
        ],
        out_shape=[
            jax.ShapeDtypeStruct((2, E, CW), jnp.float32),
            jax.ShapeDtypeStruct((E, CW), jnp.float32),
        ],
    )(gl, gr, att_row.reshape(1, D))


def _softmax_elu(sm_ref, dn_ref, bias_ref, heads, bm):
    S = jnp.concatenate([sm_ref[0], sm_ref[1]], axis=1)
    dn = dn_ref[0] + dn_ref[1]
    c = D // heads
    dnb = jnp.concatenate(
        [jnp.broadcast_to(dn[:, h:h + 1], (bm, c)) for h in range(heads)],
        axis=1) if heads > 1 else jnp.broadcast_to(dn[:, 0:1], (bm, D))
    a = S / (dnb + 1e-16) + bias_ref[...]
    return jnp.where(a > 0, a, jnp.exp(a) - 1.0)


def _combine_matmul(SM, DN, bias, Wl, bl, Wr, br, heads, bm=1000):
    n = Wl.shape[1]

    def body(sm_ref, dn_ref, bias_ref, wl_ref, blr, wr_ref, brr, ol_ref, or_ref):
        h = _softmax_elu(sm_ref, dn_ref, bias_ref, heads, bm)
        ol_ref[...] = jnp.dot(h, wl_ref[...], preferred_element_type=jnp.float32) + blr[...]
        or_ref[...] = jnp.dot(h, wr_ref[...], preferred_element_type=jnp.float32) + brr[...]

    return pl.pallas_call(
        body,
        grid=(N // bm,),
        in_specs=[
            pl.BlockSpec((2, bm, CW), lambda i: (0, i, 0)),
            pl.BlockSpec((2, bm, CW), lambda i: (0, i, 0)),
            pl.BlockSpec((1, D), lambda i: (0, 0)),
            pl.BlockSpec((D, n), lambda i: (0, 0)),
            pl.BlockSpec((1, n), lambda i: (0, 0)),
            pl.BlockSpec((D, n), lambda i: (0, 0)),
            pl.BlockSpec((1, n), lambda i: (0, 0)),
        ],
        out_specs=[
            pl.BlockSpec((bm, n), lambda i: (i, 0)),
            pl.BlockSpec((bm, n), lambda i: (i, 0)),
        ],
        out_shape=[
            jax.ShapeDtypeStruct((N, n), jnp.float32),
            jax.ShapeDtypeStruct((N, n), jnp.float32),
        ],
    )(SM, DN, bias.reshape(1, D), Wl, bl.reshape(1, n), Wr, br.reshape(1, n))


def _combine_final(SM, DN, bias, Wp, bp, heads=1, bm=1000):
    n = Wp.shape[1]

    def body(sm_ref, dn_ref, bias_ref, w_ref, br_, o_ref):
        h = _softmax_elu(sm_ref, dn_ref, bias_ref, heads, bm)
        o_ref[...] = jnp.dot(h, w_ref[...], preferred_element_type=jnp.float32) + br_[...]

    return pl.pallas_call(
        body,
        grid=(N // bm,),
        in_specs=[
            pl.BlockSpec((2, bm, CW), lambda i: (0, i, 0)),
            pl.BlockSpec((2, bm, CW), lambda i: (0, i, 0)),
            pl.BlockSpec((1, D), lambda i: (0, 0)),
            pl.BlockSpec((D, n), lambda i: (0, 0)),
            pl.BlockSpec((1, n), lambda i: (0, 0)),
        ],
        out_specs=pl.BlockSpec((bm, n), lambda i: (i, 0)),
        out_shape=jax.ShapeDtypeStruct((N, n), jnp.float32),
    )(SM, DN, bias.reshape(1, D), Wp, bp.reshape(1, n))


def _sc_gather2(xl, xr, src, dst):
    mesh = plsc.VectorSubcoreMesh(core_axis_name="c", subcore_axis_name="s")
    NW = NC * NS
    CHG = 80
    NCHUNK_G = E // CHG
    iters = (NCHUNK_G + NW - 1) // NW

    @functools.partial(
        pl.kernel,
        mesh=mesh,
        out_type=[
            jax.ShapeDtypeStruct((E, D), jnp.float32),
            jax.ShapeDtypeStruct((E, D), jnp.float32),
        ],
        scratch_types=[
            pltpu.VMEM((CHG,), jnp.int32),
            pltpu.VMEM((CHG,), jnp.int32),
            pltpu.VMEM((CHG,), jnp.int32),
            pltpu.VMEM((CHG,), jnp.int32),
            pltpu.VMEM((CHG, D), jnp.float32),
            pltpu.VMEM((CHG, D), jnp.float32),
            pltpu.VMEM((CHG, D), jnp.float32),
            pltpu.VMEM((CHG, D), jnp.float32),
            pltpu.SemaphoreType.DMA,
            pltpu.SemaphoreType.DMA,
        ],
    )
    def k(xl_hbm, xr_hbm, src_hbm, dst_hbm, gl_hbm, gr_hbm,
          ibs0, ibd0, ibs1, ibd1, rs0, rd0, rs1, rd1, sg0, sg1):
        wid = lax.axis_index("s") * NC + lax.axis_index("c")
        ib = ((ibs0, ibd0), (ibs1, ibd1))
        rb = ((rs0, rd0), (rs1, rd1))
        sg = (sg0, sg1)

        def start(slot, b):
            chunk = slot * NW + wid

            @pl.when(chunk < NCHUNK_G)
            def _():
                base = pl.multiple_of(chunk * CHG, 8)
                pltpu.sync_copy(src_hbm.at[pl.ds(base, CHG)], ib[b][0])
                pltpu.sync_copy(dst_hbm.at[pl.ds(base, CHG)], ib[b][1])
                pltpu.async_copy(xl_hbm.at[ib[b][0]], rb[b][0], sg[b])
                pltpu.async_copy(xr_hbm.at[ib[b][1]], rb[b][1], sg[b])

        def finish(slot, b):
            chunk = slot * NW + wid

            @pl.when(chunk < NCHUNK_G)
            def _():
                base = pl.multiple_of(chunk * CHG, 8)
                pltpu.make_async_copy(xl_hbm.at[ib[b][0]], rb[b][0], sg[b]).wait()
                pltpu.make_async_copy(xr_hbm.at[ib[b][1]], rb[b][1], sg[b]).wait()
                pltpu.sync_copy(rb[b][0], gl_hbm.at[pl.ds(base, CHG)])
                pltpu.sync_copy(rb[b][1], gr_hbm.at[pl.ds(base, CHG)])

        start(0, 0)

        def body(g, carry):
            start(2 * g + 1, 1)
            finish(2 * g, 0)
            start(2 * g + 2, 0)
            finish(2 * g + 1, 1)
            return carry

        lax.fori_loop(0, (iters + 1) // 2, body, 0)

    return k(xl, xr, src, dst)


def _sc_seg_sum(vals2, aux, idx):
    mesh = plsc.VectorSubcoreMesh(core_axis_name="c", subcore_axis_name="s")
    iters = (NCHUNK + NS - 1) // NS
    HALF = NCHUNK // NC
    hiters = (HALF + NS - 1) // NS
    BR = 80
    NB = N // BR
    biters = (NB + NS - 1) // NS

    @functools.partial(
        pl.kernel,
        mesh=mesh,
        out_type=[
            jax.ShapeDtypeStruct((NC, N, CW), jnp.float32),
            jax.ShapeDtypeStruct((NC, N, CW), jnp.float32),
        ],
        scratch_types=[
            pltpu.VMEM((CH,), jnp.int32),
            pltpu.VMEM((CH,), jnp.int32),
            pltpu.VMEM((CH, CW), jnp.float32),
            pltpu.VMEM((CH, CW), jnp.float32),
            pltpu.VMEM_SHARED((N, CW), jnp.float32),
            pltpu.SemaphoreType.DMA,
            pltpu.SemaphoreType.DMA,
        ],
    )
    def k(vals_hbm, aux_hbm, idx_hbm, zeros_hbm, om_hbm, od_hbm,
          ibuf0, ibuf1, vbuf0, vbuf1, acc, sv0, sv1):
        cid = lax.axis_index("c")
        sid = lax.axis_index("s")
        ibuf = (ibuf0, ibuf1)
        vbuf = (vbuf0, vbuf1)
        sv = (sv0, sv1)

        def zero_acc():
            def zero(j, carry):
                blk = j * NS + sid

                @pl.when(blk < NB)
                def _():
                    r0 = pl.multiple_of(blk * BR, 8)
                    pltpu.sync_copy(zeros_hbm, acc.at[pl.ds(r0, BR)])

                return carry

            lax.fori_loop(0, biters, zero, 0)

        def drain(dst_hbm):
            def d(j, carry):
                blk = j * NS + sid

                @pl.when(blk < NB)
                def _():
                    r0 = pl.multiple_of(blk * BR, 8)
                    pltpu.sync_copy(acc.at[pl.ds(r0, BR)],
                                    dst_hbm.at[cid].at[pl.ds(r0, BR)])

                return carry

            lax.fori_loop(0, biters, d, 0)

        def pipelined_pass(src_hbm, n_pairs, nloc, loc_to_chunk):
            def start(slot, b):
                loc = slot * NS + sid

                @pl.when(loc < nloc)
                def _():
                    base = pl.multiple_of(loc_to_chunk(loc) * CH, 8)
                    pltpu.sync_copy(idx_hbm.at[pl.ds(base, CH)], ibuf[b])
                    pltpu.async_copy(src_hbm.at[pl.ds(base, CH)], vbuf[b], sv[b])

            def finish(slot, b):
                loc = slot * NS + sid

                @pl.when(loc < nloc)
                def _():
                    base = pl.multiple_of(loc_to_chunk(loc) * CH, 8)
                    pltpu.make_async_copy(
                        src_hbm.at[pl.ds(base, CH)], vbuf[b], sv[b]).wait()
                    pltpu.sync_copy(vbuf[b], acc.at[ibuf[b]], add=True)

            start(0, 0)

            def body(g, carry):
                start(2 * g + 1, 1)
                finish(2 * g, 0)
                start(2 * g + 2, 0)
                finish(2 * g + 1, 1)
                return carry

            lax.fori_loop(0, n_pairs, body, 0)

        zero_acc()
        plsc.subcore_barrier()
        pipelined_pass(vals_hbm.at[cid], (iters + 1) // 2, NCHUNK, lambda l: l)
        plsc.subcore_barrier()
        drain(om_hbm)
        plsc.subcore_barrier()

        zero_acc()
        plsc.subcore_barrier()
        pipelined_pass(aux_hbm, hiters // 2, HALF, lambda l: cid * HALF + l)
        plsc.subcore_barrier()
        drain(od_hbm)

    return k(vals2, aux, idx, jnp.zeros((BR, CW), jnp.float32))


def kernel(x, edge_index, Wl1, bl1, Wr1, br1, att1, bias1,
           Wl2, bl2, Wr2, br2, att2, bias2, Wp, bp):
    src = edge_index[0]
    dst = edge_index[1]

    xl1, xr1 = _dual_matmul(x, Wl1, bl1, Wr1, br1)
    gl1, gr1 = _sc_gather2(xl1, xr1, src, dst)
    PW1, AUX1 = _edge_scores(gl1, gr1, att1.reshape(-1), heads=4)
    SM1, DN1 = _sc_seg_sum(PW1, AUX1, dst)

    xl2, xr2 = _combine_matmul(SM1, DN1, bias1, Wl2, bl2, Wr2, br2, heads=4)
    gl2, gr2 = _sc_gather2(xl2, xr2, src, dst)
    PW2, AUX2 = _edge_scores(gl2, gr2, att2.reshape(-1), heads=1)
    SM2, DN2 = _sc_seg_sum(PW2, AUX2, dst)

    return _combine_final(SM2, DN2, bias2, Wp, bp)


# speedup vs baseline: 16.8508x; 1.0011x over previous
import functools

import jax
import jax.numpy as jnp
from jax import lax
from jax.experimental import pallas as pl
from jax.experimental.pallas import tpu as pltpu
from jax.experimental.pallas import tpu_sc as plsc

N = 10000
E = 160000
D = 256
NC = 2
NS = 16
CW = 128
CH = 128
NCHUNK = E // CH


def _dual_matmul(x, Wl, bl, Wr, br, bm=1000):
    m, k = x.shape
    n = Wl.shape[1]

    def body(x_ref, wl_ref, blr, wr_ref, brr, ol_ref, or_ref):
        xb = x_ref[...]
        ol_ref[...] = jnp.dot(xb, wl_ref[...], preferred_element_type=jnp.float32) + blr[...]
        or_ref[...] = jnp.dot(xb, wr_ref[...], preferred_element_type=jnp.float32) + brr[...]

    return pl.pallas_call(
        body,
        grid=(m // bm,),
        in_specs=[
            pl.BlockSpec((bm, k), lambda i: (i, 0)),
            pl.BlockSpec((k, n), lambda i: (0, 0)),
            pl.BlockSpec((1, n), lambda i: (0, 0)),
            pl.BlockSpec((k, n), lambda i: (0, 0)),
            pl.BlockSpec((1, n), lambda i: (0, 0)),
        ],
        out_specs=[
            pl.BlockSpec((bm, n), lambda i: (i, 0)),
            pl.BlockSpec((bm, n), lambda i: (i, 0)),
        ],
        out_shape=[
            jax.ShapeDtypeStruct((m, n), jnp.float32),
            jax.ShapeDtypeStruct((m, n), jnp.float32),
        ],
    )(x, Wl, bl.reshape(1, n), Wr, br.reshape(1, n))


def _edge_scores(gl, gr, att_row, heads, be=2000):
    c = D // heads

    def body(gl_ref, gr_ref, att_ref, pw_ref, aux_ref):
        glb = gl_ref[...]
        z = glb + gr_ref[...]
        z = jnp.where(z >= 0, z, 0.2 * z)
        prod = z * att_ref[...]
        exps = []
        parts = []
        for h in range(heads):
            eh = jnp.exp(jnp.sum(prod[:, h * c:(h + 1) * c], axis=1, keepdims=True))
            exps.append(eh)
            parts.append(jnp.broadcast_to(eh, (be, c)))
        wex = jnp.concatenate(parts, axis=1) if heads > 1 else parts[0]
        m = wex * glb
        pw_ref[0] = m[:, :CW]
        pw_ref[1] = m[:, CW:]
        aux_ref[...] = jnp.concatenate(
            exps + [jnp.zeros((be, CW - heads), jnp.float32)], axis=1)

    return pl.pallas_call(
        body,
        grid=(E // be,),
        in_specs=[
            pl.BlockSpec((be, D), lambda i: (i, 0)),
            pl.BlockSpec((be, D), lambda i: (i, 0)),
            pl.BlockSpec((1, D), lambda i: (0, 0)),
        ],
        out_specs=[
            pl.BlockSpec((2, be, CW), lambda i: (0, i, 0)),
            pl.BlockSpec((be, CW), lambda i: (i, 0)),
        ],
        out_shape=[
            jax.ShapeDtypeStruct((2, E, CW), jnp.float32),
            jax.ShapeDtypeStruct((E, CW), jnp.float32),
        ],
    )(gl, gr, att_row.reshape(1, D))


def _softmax_elu(sm_ref, dn_ref, bias_ref, heads, bm):
    S = jnp.concatenate([sm_ref[0], sm_ref[1]], axis=1)
    dn = dn_ref[0] + dn_ref[1]
    c = D // heads
    dnb = jnp.concatenate(
        [jnp.broadcast_to(dn[:, h:h + 1], (bm, c)) for h in range(heads)],
        axis=1) if heads > 1 else jnp.broadcast_to(dn[:, 0:1], (bm, D))
    a = S / (dnb + 1e-16) + bias_ref[...]
    return jnp.where(a > 0, a, jnp.exp(a) - 1.0)


def _combine_matmul(SM, DN, bias, Wl, bl, Wr, br, heads, bm=1000):
    n = Wl.shape[1]

    def body(sm_ref, dn_ref, bias_ref, wl_ref, blr, wr_ref, brr, ol_ref, or_ref):
        h = _softmax_elu(sm_ref, dn_ref, bias_ref, heads, bm)
        ol_ref[...] = jnp.dot(h, wl_ref[...], preferred_element_type=jnp.float32) + blr[...]
        or_ref[...] = jnp.dot(h, wr_ref[...], preferred_element_type=jnp.float32) + brr[...]

    return pl.pallas_call(
        body,
        grid=(N // bm,),
        in_specs=[
            pl.BlockSpec((2, bm, CW), lambda i: (0, i, 0)),
            pl.BlockSpec((2, bm, CW), lambda i: (0, i, 0)),
            pl.BlockSpec((1, D), lambda i: (0, 0)),
            pl.BlockSpec((D, n), lambda i: (0, 0)),
            pl.BlockSpec((1, n), lambda i: (0, 0)),
            pl.BlockSpec((D, n), lambda i: (0, 0)),
            pl.BlockSpec((1, n), lambda i: (0, 0)),
        ],
        out_specs=[
            pl.BlockSpec((bm, n), lambda i: (i, 0)),
            pl.BlockSpec((bm, n), lambda i: (i, 0)),
        ],
        out_shape=[
            jax.ShapeDtypeStruct((N, n), jnp.float32),
            jax.ShapeDtypeStruct((N, n), jnp.float32),
        ],
    )(SM, DN, bias.reshape(1, D), Wl, bl.reshape(1, n), Wr, br.reshape(1, n))


def _combine_final(SM, DN, bias, Wp, bp, heads=1, bm=1000):
    n = Wp.shape[1]

    def body(sm_ref, dn_ref, bias_ref, w_ref, br_, o_ref):
        h = _softmax_elu(sm_ref, dn_ref, bias_ref, heads, bm)
        o_ref[...] = jnp.dot(h, w_ref[...], preferred_element_type=jnp.float32) + br_[...]

    return pl.pallas_call(
        body,
        grid=(N // bm,),
        in_specs=[
            pl.BlockSpec((2, bm, CW), lambda i: (0, i, 0)),
            pl.BlockSpec((2, bm, CW), lambda i: (0, i, 0)),
            pl.BlockSpec((1, D), lambda i: (0, 0)),
            pl.BlockSpec((D, n), lambda i: (0, 0)),
            pl.BlockSpec((1, n), lambda i: (0, 0)),
        ],
        out_specs=pl.BlockSpec((bm, n), lambda i: (i, 0)),
        out_shape=jax.ShapeDtypeStruct((N, n), jnp.float32),
    )(SM, DN, bias.reshape(1, D), Wp, bp.reshape(1, n))


def _sc_gather2(xl, xr, src, dst):
    mesh = plsc.VectorSubcoreMesh(core_axis_name="c", subcore_axis_name="s")
    NW = NC * NS
    CHG = 80
    NCHUNK_G = E // CHG
    iters = (NCHUNK_G + NW - 1) // NW

    @functools.partial(
        pl.kernel,
        mesh=mesh,
        out_type=[
            jax.ShapeDtypeStruct((E, D), jnp.float32),
            jax.ShapeDtypeStruct((E, D), jnp.float32),
        ],
        scratch_types=[
            pltpu.VMEM((CHG,), jnp.int32),
            pltpu.VMEM((CHG,), jnp.int32),
            pltpu.VMEM((CHG,), jnp.int32),
            pltpu.VMEM((CHG,), jnp.int32),
            pltpu.VMEM((CHG, D), jnp.float32),
            pltpu.VMEM((CHG, D), jnp.float32),
            pltpu.VMEM((CHG, D), jnp.float32),
            pltpu.VMEM((CHG, D), jnp.float32),
            pltpu.SemaphoreType.DMA,
            pltpu.SemaphoreType.DMA,
            pltpu.SemaphoreType.DMA,
            pltpu.SemaphoreType.DMA,
        ],
    )
    def k(xl_hbm, xr_hbm, src_hbm, dst_hbm, gl_hbm, gr_hbm,
          ibs0, ibd0, ibs1, ibd1, rs0, rd0, rs1, rd1, sg0, sg1, sw0, sw1):
        wid = lax.axis_index("s") * NC + lax.axis_index("c")
        ib = ((ibs0, ibd0), (ibs1, ibd1))
        rb = ((rs0, rd0), (rs1, rd1))
        sg = (sg0, sg1)
        sw = (sw0, sw1)

        def start(slot, b):
            chunk = slot * NW + wid
            prev = chunk - 2 * NW

            @pl.when((prev >= 0) & (prev < NCHUNK_G))
            def _():
                pbase = pl.multiple_of(prev * CHG, 8)
                pltpu.make_async_copy(
                    rb[b][0], gl_hbm.at[pl.ds(pbase, CHG)], sw[b]).wait()
                pltpu.make_async_copy(
                    rb[b][1], gr_hbm.at[pl.ds(pbase, CHG)], sw[b]).wait()

            @pl.when(chunk < NCHUNK_G)
            def _():
                base = pl.multiple_of(chunk * CHG, 8)
                pltpu.sync_copy(src_hbm.at[pl.ds(base, CHG)], ib[b][0])
                pltpu.sync_copy(dst_hbm.at[pl.ds(base, CHG)], ib[b][1])
                pltpu.async_copy(xl_hbm.at[ib[b][0]], rb[b][0], sg[b])
                pltpu.async_copy(xr_hbm.at[ib[b][1]], rb[b][1], sg[b])

        def finish(slot, b):
            chunk = slot * NW + wid

            @pl.when(chunk < NCHUNK_G)
            def _():
                base = pl.multiple_of(chunk * CHG, 8)
                pltpu.make_async_copy(xl_hbm.at[ib[b][0]], rb[b][0], sg[b]).wait()
                pltpu.make_async_copy(xr_hbm.at[ib[b][1]], rb[b][1], sg[b]).wait()
                pltpu.async_copy(rb[b][0], gl_hbm.at[pl.ds(base, CHG)], sw[b])
                pltpu.async_copy(rb[b][1], gr_hbm.at[pl.ds(base, CHG)], sw[b])

        start(0, 0)

        def body(g, carry):
            start(2 * g + 1, 1)
            finish(2 * g, 0)
            start(2 * g + 2, 0)
            finish(2 * g + 1, 1)
            return carry

        lax.fori_loop(0, (iters + 1) // 2, body, 0)

    return k(xl, xr, src, dst)


def _sc_seg_sum(vals2, aux, idx):
    mesh = plsc.VectorSubcoreMesh(core_axis_name="c", subcore_axis_name="s")
    iters = (NCHUNK + NS - 1) // NS
    HALF = NCHUNK // NC
    hiters = (HALF + NS - 1) // NS
    BR = 80
    NB = N // BR
    biters = (NB + NS - 1) // NS

    @functools.partial(
        pl.kernel,
        mesh=mesh,
        out_type=[
            jax.ShapeDtypeStruct((NC, N, CW), jnp.float32),
            jax.ShapeDtypeStruct((NC, N, CW), jnp.float32),
        ],
        scratch_types=[
            pltpu.VMEM((CH,), jnp.int32),
            pltpu.VMEM((CH,), jnp.int32),
            pltpu.VMEM((CH, CW), jnp.float32),
            pltpu.VMEM((CH, CW), jnp.float32),
            pltpu.VMEM_SHARED((N, CW), jnp.float32),
            pltpu.SemaphoreType.DMA,
            pltpu.SemaphoreType.DMA,
            pltpu.SemaphoreType.DMA,
            pltpu.SemaphoreType.DMA,
        ],
    )
    def k(vals_hbm, aux_hbm, idx_hbm, zeros_hbm, om_hbm, od_hbm,
          ibuf0, ibuf1, vbuf0, vbuf1, acc, sv0, sv1, sa0, sa1):
        cid = lax.axis_index("c")
        sid = lax.axis_index("s")
        ibuf = (ibuf0, ibuf1)
        vbuf = (vbuf0, vbuf1)
        sv = (sv0, sv1)
        sa = (sa0, sa1)

        def zero_acc():
            def zero(j, carry):
                blk = j * NS + sid

                @pl.when(blk < NB)
                def _():
                    r0 = pl.multiple_of(blk * BR, 8)
                    pltpu.sync_copy(zeros_hbm, acc.at[pl.ds(r0, BR)])

                return carry

            lax.fori_loop(0, biters, zero, 0)

        def drain(dst_hbm):
            def d(j, carry):
                blk = j * NS + sid

                @pl.when(blk < NB)
                def _():
                    r0 = pl.multiple_of(blk * BR, 8)
                    pltpu.sync_copy(acc.at[pl.ds(r0, BR)],
                                    dst_hbm.at[cid].at[pl.ds(r0, BR)])

                return carry

            lax.fori_loop(0, biters, d, 0)

        def pipelined_pass(src_hbm, n_pairs, nloc, loc_to_chunk):
            def drain_add(slot, b):
                prev = (slot - 2) * NS + sid

                @pl.when((prev >= 0) & (prev < nloc))
                def _():
                    pltpu.make_async_copy(
                        vbuf[b], acc.at[ibuf[b]], sa[b]).wait()

            def start(slot, b):
                loc = slot * NS + sid
                drain_add(slot, b)

                @pl.when(loc < nloc)
                def _():
                    base = pl.multiple_of(loc_to_chunk(loc) * CH, 8)
                    pltpu.sync_copy(idx_hbm.at[pl.ds(base, CH)], ibuf[b])
                    pltpu.async_copy(src_hbm.at[pl.ds(base, CH)], vbuf[b], sv[b])

            def finish(slot, b):
                loc = slot * NS + sid

                @pl.when(loc < nloc)
                def _():
                    base = pl.multiple_of(loc_to_chunk(loc) * CH, 8)
                    pltpu.make_async_copy(
                        src_hbm.at[pl.ds(base, CH)], vbuf[b], sv[b]).wait()
                    pltpu.async_copy(vbuf[b], acc.at[ibuf[b]], sa[b], add=True)

            start(0, 0)

            def body(g, carry):
                start(2 * g + 1, 1)
                finish(2 * g, 0)
                start(2 * g + 2, 0)
                finish(2 * g + 1, 1)
                return carry

            lax.fori_loop(0, n_pairs, body, 0)
            drain_add(2 * n_pairs + 1, 1)
            drain_add(2 * n_pairs + 2, 0)

        zero_acc()
        plsc.subcore_barrier()
        pipelined_pass(vals_hbm.at[cid], (iters + 1) // 2, NCHUNK, lambda l: l)
        plsc.subcore_barrier()
        drain(om_hbm)
        plsc.subcore_barrier()

        zero_acc()
        plsc.subcore_barrier()
        pipelined_pass(aux_hbm, hiters // 2, HALF, lambda l: cid * HALF + l)
        plsc.subcore_barrier()
        drain(od_hbm)

    return k(vals2, aux, idx, jnp.zeros((BR, CW), jnp.float32))


def kernel(x, edge_index, Wl1, bl1, Wr1, br1, att1, bias1,
           Wl2, bl2, Wr2, br2, att2, bias2, Wp, bp):
    src = edge_index[0]
    dst = edge_index[1]

    xl1, xr1 = _dual_matmul(x, Wl1, bl1, Wr1, br1)
    gl1, gr1 = _sc_gather2(xl1, xr1, src, dst)
    PW1, AUX1 = _edge_scores(gl1, gr1, att1.reshape(-1), heads=4)
    SM1, DN1 = _sc_seg_sum(PW1, AUX1, dst)

    xl2, xr2 = _combine_matmul(SM1, DN1, bias1, Wl2, bl2, Wr2, br2, heads=4)
    gl2, gr2 = _sc_gather2(xl2, xr2, src, dst)
    PW2, AUX2 = _edge_scores(gl2, gr2, att2.reshape(-1), heads=1)
    SM2, DN2 = _sc_seg_sum(PW2, AUX2, dst)

    return _combine_final(SM2, DN2, bias2, Wp, bp)


# speedup vs baseline: 17.2111x; 1.0214x over previous
import functools

import jax
import jax.numpy as jnp
from jax import lax
from jax.experimental import pallas as pl
from jax.experimental.pallas import tpu as pltpu
from jax.experimental.pallas import tpu_sc as plsc

N = 10000
E = 160000
D = 256
NC = 2
NS = 16
CW = 128
CH = 128
NCHUNK = E // CH


def _dual_matmul(x, Wl, bl, Wr, br, bm=1000):
    m, k = x.shape
    n = Wl.shape[1]

    def body(x_ref, wl_ref, blr, wr_ref, brr, ol_ref, or_ref):
        xb = x_ref[...]
        ol_ref[...] = jnp.dot(xb, wl_ref[...], preferred_element_type=jnp.float32) + blr[...]
        or_ref[...] = jnp.dot(xb, wr_ref[...], preferred_element_type=jnp.float32) + brr[...]

    return pl.pallas_call(
        body,
        grid=(m // bm,),
        in_specs=[
            pl.BlockSpec((bm, k), lambda i: (i, 0)),
            pl.BlockSpec((k, n), lambda i: (0, 0)),
            pl.BlockSpec((1, n), lambda i: (0, 0)),
            pl.BlockSpec((k, n), lambda i: (0, 0)),
            pl.BlockSpec((1, n), lambda i: (0, 0)),
        ],
        out_specs=[
            pl.BlockSpec((bm, n), lambda i: (i, 0)),
            pl.BlockSpec((bm, n), lambda i: (i, 0)),
        ],
        out_shape=[
            jax.ShapeDtypeStruct((m, n), jnp.float32),
            jax.ShapeDtypeStruct((m, n), jnp.float32),
        ],
    )(x, Wl, bl.reshape(1, n), Wr, br.reshape(1, n))


def _edge_scores(gl, gr, att_row, heads, be=2000):
    c = D // heads

    def body(gl_ref, gr_ref, att_ref, pw_ref, aux_ref):
        glb = gl_ref[...]
        z = glb + gr_ref[...]
        z = jnp.where(z >= 0, z, 0.2 * z)
        prod = z * att_ref[...]
        exps = []
        parts = []
        for h in range(heads):
            eh = jnp.exp(jnp.sum(prod[:, h * c:(h + 1) * c], axis=1, keepdims=True))
            exps.append(eh)
            parts.append(jnp.broadcast_to(eh, (be, c)))
        wex = jnp.concatenate(parts, axis=1) if heads > 1 else parts[0]
        m = wex * glb
        pw_ref[0] = m[:, :CW]
        pw_ref[1] = m[:, CW:]
        aux_ref[...] = jnp.concatenate(
            exps + [jnp.zeros((be, CW - heads), jnp.float32)], axis=1)

    ne = gl.shape[0]
    return pl.pallas_call(
        body,
        grid=(ne // be,),
        in_specs=[
            pl.BlockSpec((be, D), lambda i: (i, 0)),
            pl.BlockSpec((be, D), lambda i: (i, 0)),
            pl.BlockSpec((1, D), lambda i: (0, 0)),
        ],
        out_specs=[
            pl.BlockSpec((2, be, CW), lambda i: (0, i, 0)),
            pl.BlockSpec((be, CW), lambda i: (i, 0)),
        ],
        out_shape=[
            jax.ShapeDtypeStruct((2, ne, CW), jnp.float32),
            jax.ShapeDtypeStruct((ne, CW), jnp.float32),
        ],
    )(gl, gr, att_row.reshape(1, D))


def _softmax_elu(sm_ref, dn_ref, bias_ref, heads, bm):
    S = jnp.concatenate([sm_ref[0], sm_ref[1]], axis=1)
    dn = dn_ref[0] + dn_ref[1]
    c = D // heads
    dnb = jnp.concatenate(
        [jnp.broadcast_to(dn[:, h:h + 1], (bm, c)) for h in range(heads)],
        axis=1) if heads > 1 else jnp.broadcast_to(dn[:, 0:1], (bm, D))
    a = S / (dnb + 1e-16) + bias_ref[...]
    return jnp.where(a > 0, a, jnp.exp(a) - 1.0)


def _combine_matmul(SM, DN, bias, Wl, bl, Wr, br, heads, bm=1000):
    n = Wl.shape[1]

    def body(sm_ref, dn_ref, bias_ref, wl_ref, blr, wr_ref, brr, ol_ref, or_ref):
        h = _softmax_elu(sm_ref, dn_ref, bias_ref, heads, bm)
        ol_ref[...] = jnp.dot(h, wl_ref[...], preferred_element_type=jnp.float32) + blr[...]
        or_ref[...] = jnp.dot(h, wr_ref[...], preferred_element_type=jnp.float32) + brr[...]

    return pl.pallas_call(
        body,
        grid=(N // bm,),
        in_specs=[
            pl.BlockSpec((2, bm, CW), lambda i: (0, i, 0)),
            pl.BlockSpec((2, bm, CW), lambda i: (0, i, 0)),
            pl.BlockSpec((1, D), lambda i: (0, 0)),
            pl.BlockSpec((D, n), lambda i: (0, 0)),
            pl.BlockSpec((1, n), lambda i: (0, 0)),
            pl.BlockSpec((D, n), lambda i: (0, 0)),
            pl.BlockSpec((1, n), lambda i: (0, 0)),
        ],
        out_specs=[
            pl.BlockSpec((bm, n), lambda i: (i, 0)),
            pl.BlockSpec((bm, n), lambda i: (i, 0)),
        ],
        out_shape=[
            jax.ShapeDtypeStruct((N, n), jnp.float32),
            jax.ShapeDtypeStruct((N, n), jnp.float32),
        ],
    )(SM, DN, bias.reshape(1, D), Wl, bl.reshape(1, n), Wr, br.reshape(1, n))


def _combine_final(SM, DN, bias, Wp, bp, heads=1, bm=1000):
    n = Wp.shape[1]

    def body(sm_ref, dn_ref, bias_ref, w_ref, br_, o_ref):
        h = _softmax_elu(sm_ref, dn_ref, bias_ref, heads, bm)
        o_ref[...] = jnp.dot(h, w_ref[...], preferred_element_type=jnp.float32) + br_[...]

    return pl.pallas_call(
        body,
        grid=(N // bm,),
        in_specs=[
            pl.BlockSpec((2, bm, CW), lambda i: (0, i, 0)),
            pl.BlockSpec((2, bm, CW), lambda i: (0, i, 0)),
            pl.BlockSpec((1, D), lambda i: (0, 0)),
            pl.BlockSpec((D, n), lambda i: (0, 0)),
            pl.BlockSpec((1, n), lambda i: (0, 0)),
        ],
        out_specs=pl.BlockSpec((bm, n), lambda i: (i, 0)),
        out_shape=jax.ShapeDtypeStruct((N, n), jnp.float32),
    )(SM, DN, bias.reshape(1, D), Wp, bp.reshape(1, n))


def _sc_gather2(xl, xr, src, dst):
    ne = src.shape[0]
    mesh = plsc.VectorSubcoreMesh(core_axis_name="c", subcore_axis_name="s")
    NW = NC * NS
    CHG = 80
    NCHUNK_G = ne // CHG
    iters = (NCHUNK_G + NW - 1) // NW

    @functools.partial(
        pl.kernel,
        mesh=mesh,
        out_type=[
            jax.ShapeDtypeStruct((ne, D), jnp.float32),
            jax.ShapeDtypeStruct((ne, D), jnp.float32),
        ],
        scratch_types=[
            pltpu.VMEM((CHG,), jnp.int32),
            pltpu.VMEM((CHG,), jnp.int32),
            pltpu.VMEM((CHG,), jnp.int32),
            pltpu.VMEM((CHG,), jnp.int32),
            pltpu.VMEM((CHG, D), jnp.float32),
            pltpu.VMEM((CHG, D), jnp.float32),
            pltpu.VMEM((CHG, D), jnp.float32),
            pltpu.VMEM((CHG, D), jnp.float32),
            pltpu.SemaphoreType.DMA,
            pltpu.SemaphoreType.DMA,
            pltpu.SemaphoreType.DMA,
            pltpu.SemaphoreType.DMA,
        ],
    )
    def k(xl_hbm, xr_hbm, src_hbm, dst_hbm, gl_hbm, gr_hbm,
          ibs0, ibd0, ibs1, ibd1, rs0, rd0, rs1, rd1, sg0, sg1, sw0, sw1):
        wid = lax.axis_index("s") * NC + lax.axis_index("c")
        ib = ((ibs0, ibd0), (ibs1, ibd1))
        rb = ((rs0, rd0), (rs1, rd1))
        sg = (sg0, sg1)
        sw = (sw0, sw1)

        def drain_w(slot, b):
            prev = (slot - 2) * NW + wid

            @pl.when((prev >= 0) & (prev < NCHUNK_G))
            def _():
                pbase = pl.multiple_of(prev * CHG, 8)
                pltpu.make_async_copy(
                    rb[b][0], gl_hbm.at[pl.ds(pbase, CHG)], sw[b]).wait()
                pltpu.make_async_copy(
                    rb[b][1], gr_hbm.at[pl.ds(pbase, CHG)], sw[b]).wait()

        def start(slot, b):
            chunk = slot * NW + wid
            drain_w(slot, b)

            @pl.when(chunk < NCHUNK_G)
            def _():
                base = pl.multiple_of(chunk * CHG, 8)
                pltpu.sync_copy(src_hbm.at[pl.ds(base, CHG)], ib[b][0])
                pltpu.sync_copy(dst_hbm.at[pl.ds(base, CHG)], ib[b][1])
                pltpu.async_copy(xl_hbm.at[ib[b][0]], rb[b][0], sg[b])
                pltpu.async_copy(xr_hbm.at[ib[b][1]], rb[b][1], sg[b])

        def finish(slot, b):
            chunk = slot * NW + wid

            @pl.when(chunk < NCHUNK_G)
            def _():
                base = pl.multiple_of(chunk * CHG, 8)
                pltpu.make_async_copy(xl_hbm.at[ib[b][0]], rb[b][0], sg[b]).wait()
                pltpu.make_async_copy(xr_hbm.at[ib[b][1]], rb[b][1], sg[b]).wait()
                pltpu.async_copy(rb[b][0], gl_hbm.at[pl.ds(base, CHG)], sw[b])
                pltpu.async_copy(rb[b][1], gr_hbm.at[pl.ds(base, CHG)], sw[b])

        start(0, 0)

        def body(g, carry):
            start(2 * g + 1, 1)
            finish(2 * g, 0)
            start(2 * g + 2, 0)
            finish(2 * g + 1, 1)
            return carry

        npairs = (iters + 1) // 2
        lax.fori_loop(0, npairs, body, 0)
        drain_w(2 * npairs + 1, 1)
        drain_w(2 * npairs + 2, 0)

    return k(xl, xr, src, dst)


def _sc_seg_sum(va, vb, aux_a, aux_b, idx):
    mesh = plsc.VectorSubcoreMesh(core_axis_name="c", subcore_axis_name="s")
    iters = (NCHUNK + NS - 1) // NS
    HALF = NCHUNK // NC
    hiters = (HALF + NS - 1) // NS
    EH = E // NC
    BR = 80
    NB = N // BR
    biters = (NB + NS - 1) // NS

    @functools.partial(
        pl.kernel,
        mesh=mesh,
        out_type=[
            jax.ShapeDtypeStruct((NC, N, CW), jnp.float32),
            jax.ShapeDtypeStruct((NC, N, CW), jnp.float32),
        ],
        scratch_types=[
            pltpu.VMEM((CH,), jnp.int32),
            pltpu.VMEM((CH,), jnp.int32),
            pltpu.VMEM((CH, CW), jnp.float32),
            pltpu.VMEM((CH, CW), jnp.float32),
            pltpu.VMEM_SHARED((N, CW), jnp.float32),
            pltpu.SemaphoreType.DMA,
            pltpu.SemaphoreType.DMA,
            pltpu.SemaphoreType.DMA,
            pltpu.SemaphoreType.DMA,
        ],
    )
    def k(va_hbm, vb_hbm, auxa_hbm, auxb_hbm, idx_hbm, zeros_hbm, om_hbm,
          od_hbm, ibuf0, ibuf1, vbuf0, vbuf1, acc, sv0, sv1, sa0, sa1):
        cid = lax.axis_index("c")
        sid = lax.axis_index("s")
        ibuf = (ibuf0, ibuf1)
        vbuf = (vbuf0, vbuf1)
        sv = (sv0, sv1)
        sa = (sa0, sa1)

        def zero_acc():
            def zero(j, carry):
                blk = j * NS + sid

                @pl.when(blk < NB)
                def _():
                    r0 = pl.multiple_of(blk * BR, 8)
                    pltpu.sync_copy(zeros_hbm, acc.at[pl.ds(r0, BR)])

                return carry

            lax.fori_loop(0, biters, zero, 0)

        def drain(dst_hbm):
            def d(j, carry):
                blk = j * NS + sid

                @pl.when(blk < NB)
                def _():
                    r0 = pl.multiple_of(blk * BR, 8)
                    pltpu.sync_copy(acc.at[pl.ds(r0, BR)],
                                    dst_hbm.at[cid].at[pl.ds(r0, BR)])

                return carry

            lax.fori_loop(0, biters, d, 0)

        def pipelined_pass(vsrc, n_pairs, nloc, loc_to_chunk):
            def drain_add(slot, b):
                prev = (slot - 2) * NS + sid

                @pl.when((prev >= 0) & (prev < nloc))
                def _():
                    pltpu.make_async_copy(
                        vbuf[b], acc.at[ibuf[b]], sa[b]).wait()

            def start(slot, b):
                loc = slot * NS + sid
                drain_add(slot, b)

                @pl.when(loc < nloc)
                def _():
                    ibase = pl.multiple_of(loc_to_chunk(loc) * CH, 8)
                    pltpu.sync_copy(idx_hbm.at[pl.ds(ibase, CH)], ibuf[b])
                    for pred, ref, lbase in vsrc:
                        @pl.when(pred(loc))
                        def _(ref=ref, lbase=lbase):
                            base = pl.multiple_of(lbase(loc) * CH, 8)
                            pltpu.async_copy(ref.at[pl.ds(base, CH)],
                                             vbuf[b], sv[b])

            def finish(slot, b):
                loc = slot * NS + sid

                @pl.when(loc < nloc)
                def _():
                    for pred, ref, lbase in vsrc:
                        @pl.when(pred(loc))
                        def _(ref=ref, lbase=lbase):
                            base = pl.multiple_of(lbase(loc) * CH, 8)
                            pltpu.make_async_copy(
                                ref.at[pl.ds(base, CH)], vbuf[b], sv[b]).wait()
                    pltpu.async_copy(vbuf[b], acc.at[ibuf[b]], sa[b], add=True)

            start(0, 0)

            def body(g, carry):
                start(2 * g + 1, 1)
                finish(2 * g, 0)
                start(2 * g + 2, 0)
                finish(2 * g + 1, 1)
                return carry

            lax.fori_loop(0, n_pairs, body, 0)
            drain_add(2 * n_pairs + 1, 1)
            drain_add(2 * n_pairs + 2, 0)

        zero_acc()
        plsc.subcore_barrier()
        pipelined_pass(
            [(lambda l: l < HALF, va_hbm.at[cid], lambda l: l),
             (lambda l: l >= HALF, vb_hbm.at[cid], lambda l: l - HALF)],
            (iters + 1) // 2, NCHUNK, lambda l: l)
        plsc.subcore_barrier()
        drain(om_hbm)
        plsc.subcore_barrier()

        zero_acc()
        plsc.subcore_barrier()
        pipelined_pass(
            [(lambda l: cid == 0, auxa_hbm, lambda l: l),
             (lambda l: cid == 1, auxb_hbm, lambda l: l)],
            hiters // 2, HALF, lambda l: cid * HALF + l)
        plsc.subcore_barrier()
        drain(od_hbm)

    return k(va, vb, aux_a, aux_b, idx, jnp.zeros((BR, CW), jnp.float32))


def kernel(x, edge_index, Wl1, bl1, Wr1, br1, att1, bias1,
           Wl2, bl2, Wr2, br2, att2, bias2, Wp, bp):
    src = edge_index[0]
    dst = edge_index[1]
    eh = E // 2
    sa_, sb_ = src[:eh], src[eh:]
    da_, db_ = dst[:eh], dst[eh:]


    xl1, xr1 = _dual_matmul(x, Wl1, bl1, Wr1, br1)
    gl1a, gr1a = _sc_gather2(xl1, xr1, sa_, da_)
    gl1b, gr1b = _sc_gather2(xl1, xr1, sb_, db_)
    PW1a, AUX1a = _edge_scores(gl1a, gr1a, att1.reshape(-1), heads=4)
    PW1b, AUX1b = _edge_scores(gl1b, gr1b, att1.reshape(-1), heads=4)
    SM1, DN1 = _sc_seg_sum(PW1a, PW1b, AUX1a, AUX1b, dst)

    xl2, xr2 = _combine_matmul(SM1, DN1, bias1, Wl2, bl2, Wr2, br2, heads=4)
    gl2a, gr2a = _sc_gather2(xl2, xr2, sa_, da_)
    gl2b, gr2b = _sc_gather2(xl2, xr2, sb_, db_)
    PW2a, AUX2a = _edge_scores(gl2a, gr2a, att2.reshape(-1), heads=1)
    PW2b, AUX2b = _edge_scores(gl2b, gr2b, att2.reshape(-1), heads=1)
    SM2, DN2 = _sc_seg_sum(PW2a, PW2b, AUX2a, AUX2b, dst)

    return _combine_final(SM2, DN2, bias2, Wp, bp)
